# baseline (speedup 1.0000x reference)
import functools

import jax
import jax.numpy as jnp
from jax import lax
from jax.experimental import pallas as pl
from jax.experimental.pallas import tpu as pltpu
from jax.experimental.pallas import tpu_sc as plsc

_NUM_CORES = 2
_NUM_SUBCORES = 16
_NUM_WORKERS = _NUM_CORES * _NUM_SUBCORES
_LANES = 16
_CHUNK = 128


@jax.jit
def _lr_pooled_lookup(xt, table, bias16):
    F, NW, NJ, _one, C = xt.shape
    bpw = NJ * C
    L = F * bpw
    B = NW * bpw
    mesh = plsc.VectorSubcoreMesh(core_axis_name="c", subcore_axis_name="s")

    @functools.partial(
        pl.kernel,
        out_type=jax.ShapeDtypeStruct((B,), jnp.float32),
        mesh=mesh,
        scratch_types=[
            pltpu.VMEM((1, 1, L), jnp.int32),
            pltpu.VMEM((1, 1, L), jnp.float32),
            pltpu.VMEM((_LANES,), jnp.float32),
            pltpu.VMEM((bpw,), jnp.float32),
            pltpu.SemaphoreType.DMA,
            pltpu.SemaphoreType.DMA,
        ],
    )
    def k(xt_hbm, tab_hbm, bias_hbm, out_hbm, xt_v, vals_v, bias_v, acc_v, gsem, ssem):
        wid = lax.axis_index("s") * _NUM_CORES + lax.axis_index("c")
        base = wid * bpw
        stages = [
            pltpu.async_copy(
                xt_hbm.at[f, wid, j, 0],
                xt_v.at[0, 0, pl.ds((f * NJ + j) * C, C)],
                ssem,
            )
            for f in range(F)
            for j in range(NJ)
        ]
        pltpu.sync_copy(bias_hbm, bias_v)
        for s in stages:
            s.wait()
        pltpu.async_copy(tab_hbm.at[xt_v.at[0]], vals_v.at[0], gsem).wait()
        bvec = bias_v[...]
        per_chunk = C // _LANES
        for i in range(bpw // _LANES):
            j, off = i // per_chunk, (i % per_chunk) * _LANES
            acc = bvec
            for f in range(F):
                acc = acc + vals_v[0, 0, pl.ds((f * NJ + j) * C + off, _LANES)]
            acc_v[pl.ds(i * _LANES, _LANES)] = acc
        pltpu.sync_copy(acc_v, out_hbm.at[pl.ds(base, bpw)])

    return k(xt, table, bias16)


def kernel(X, table, bias):
    B, F = X.shape
    bpw = B // _NUM_WORKERS
    NJ = bpw // _CHUNK
    xt = X.T.reshape(F, _NUM_WORKERS, NJ, 1, _CHUNK)
    out = _lr_pooled_lookup(xt, table.reshape(1, -1), jnp.broadcast_to(bias, (_LANES,)))
    return out.reshape(B, 1)


# speedup vs baseline: 1.1313x; 1.1313x over previous
import functools

import jax
import jax.numpy as jnp
from jax import lax
from jax.experimental import pallas as pl
from jax.experimental.pallas import tpu as pltpu
from jax.experimental.pallas import tpu_sc as plsc

_NUM_CORES = 2
_NUM_SUBCORES = 16
_NUM_WORKERS = _NUM_CORES * _NUM_SUBCORES
_LANES = 16
_CHUNK = 128


@functools.partial(jax.jit, static_argnums=(3,))
def _lr_pooled_lookup(xt, table, bias16, F):
    NW, _one, L = xt.shape
    C = _CHUNK
    bpw = L // F
    NJ = bpw // C
    B = NW * bpw
    mesh = plsc.VectorSubcoreMesh(core_axis_name="c", subcore_axis_name="s")

    @functools.partial(
        pl.kernel,
        out_type=jax.ShapeDtypeStruct((B,), jnp.float32),
        mesh=mesh,
        compiler_params=pltpu.CompilerParams(skip_device_barrier=True),
        scratch_types=[
            pltpu.VMEM((1, 1, L), jnp.int32),
            pltpu.VMEM((1, 1, L), jnp.float32),
            pltpu.VMEM((_LANES,), jnp.float32),
            pltpu.VMEM((bpw,), jnp.float32),
            pltpu.SemaphoreType.DMA,
        ],
    )
    def k(xt_hbm, tab_hbm, bias_hbm, out_hbm, xt_v, vals_v, bias_s, acc_v, gsem):
        wid = lax.axis_index("s") * _NUM_CORES + lax.axis_index("c")
        base = wid * bpw
        pltpu.sync_copy(xt_hbm.at[wid], xt_v.at[0])
        pltpu.sync_copy(bias_hbm, bias_s.at[pl.ds(0, 1)])
        pltpu.async_copy(tab_hbm.at[xt_v.at[0]], vals_v.at[0], gsem).wait()
        bvec = jnp.full((_LANES,), bias_s[...][0], jnp.float32)
        per_chunk = C // _LANES
        for i in range(bpw // _LANES):
            j, off = i // per_chunk, (i % per_chunk) * _LANES
            acc = bvec
            for f in range(F):
                acc = acc + vals_v[0, 0, pl.ds((f * NJ + j) * C + off, _LANES)]
            acc_v[pl.ds(i * _LANES, _LANES)] = acc
        pltpu.sync_copy(acc_v, out_hbm.at[pl.ds(base, bpw)])

    return k(xt, table, bias16)


def kernel(X, table, bias):
    B, F = X.shape
    bpw = B // _NUM_WORKERS
    NJ = bpw // _CHUNK
    xt = (
        X.T.reshape(F, _NUM_WORKERS, NJ, _CHUNK)
        .swapaxes(0, 1)
        .reshape(_NUM_WORKERS, 1, F * NJ * _CHUNK)
    )
    out = _lr_pooled_lookup(xt, table.reshape(1, -1), bias, F)
    return out.reshape(B, 1)
